# Initial kernel scaffold; baseline (speedup 1.0000x reference)
import jax
import jax.numpy as jnp
from jax import lax
from jax.experimental import pallas as pl
from jax.experimental.pallas import tpu as pltpu
from jax.experimental.pallas import tpu_sc as plsc

_NC = 2
_NS = 16
_NW = _NC * _NS
_CHUNK = 128


def _gather_body(table_hbm, idx_hbm, out_hbm, idx_v, rows_v, sem):
    wid = lax.axis_index("s") * _NC + lax.axis_index("c")
    n_chunks = idx_v.shape[0]
    b_per_w = n_chunks * _CHUNK
    base = wid * b_per_w
    pltpu.sync_copy(idx_hbm.at[wid], idx_v)

    def body(g, carry):
        pltpu.async_copy(table_hbm.at[idx_v.at[g]], rows_v, sem).wait()
        pltpu.sync_copy(rows_v, out_hbm.at[pl.ds(base + g * _CHUNK, _CHUNK)])
        return carry

    lax.fori_loop(0, n_chunks, body, 0)


def kernel(data, indices):
    n, k = indices.shape
    d = data.shape[1]
    b = n * k
    n_chunks = b // (_NW * _CHUNK)
    idx3 = indices.reshape(_NW, n_chunks, _CHUNK).astype(jnp.int32)
    mesh = plsc.VectorSubcoreMesh(core_axis_name="c", subcore_axis_name="s")
    out = pl.kernel(
        _gather_body,
        out_type=jax.ShapeDtypeStruct((b, d), jnp.float32),
        mesh=mesh,
        scratch_types=[
            pltpu.VMEM((n_chunks, _CHUNK), jnp.int32),
            pltpu.VMEM((_CHUNK, d), jnp.float32),
            pltpu.SemaphoreType.DMA,
        ],
    )(data, idx3)
    return out.reshape(n, k, d)


# speedup vs baseline: 1.6847x; 1.6847x over previous
import jax
import jax.numpy as jnp
from jax import lax
from jax.experimental import pallas as pl
from jax.experimental.pallas import tpu as pltpu
from jax.experimental.pallas import tpu_sc as plsc

_NC = 2
_NS = 16
_NW = _NC * _NS
_CHUNK = 128


def _gather_body(table_hbm, idx_hbm, out_hbm, idx_v, rows_v, sem):
    wid = lax.axis_index("s") * _NC + lax.axis_index("c")
    n_chunks = idx_v.shape[0]
    b_per_w = n_chunks * _CHUNK
    base = wid * b_per_w
    pltpu.sync_copy(idx_hbm.at[wid], idx_v)

    def body(g, carry):
        pltpu.async_copy(table_hbm.at[idx_v.at[g]], rows_v, sem).wait()
        pltpu.sync_copy(rows_v, out_hbm.at[pl.ds(base + g * _CHUNK, _CHUNK)])
        return carry

    lax.fori_loop(0, n_chunks, body, 0)


def kernel(data, indices):
    n, k = indices.shape
    d = data.shape[1]
    b = n * k
    n_chunks = b // (_NW * _CHUNK)
    idx3 = indices.reshape(_NW, n_chunks, _CHUNK).astype(jnp.int32)
    mesh = plsc.VectorSubcoreMesh(core_axis_name="c", subcore_axis_name="s")
    out = pl.kernel(
        _gather_body,
        out_type=jax.ShapeDtypeStruct((b, d), jnp.float32),
        mesh=mesh,
        scratch_types=[
            pltpu.VMEM((n_chunks, _CHUNK), jnp.int32),
            pltpu.VMEM((_CHUNK, d), jnp.float32),
            pltpu.SemaphoreType.DMA,
        ],
        compiler_params=pltpu.CompilerParams(use_tc_tiling_on_sc=False),
    )(data, idx3)
    return out.reshape(n, k, d)


# speedup vs baseline: 1.8776x; 1.1145x over previous
import jax
import jax.numpy as jnp
from jax import lax
from jax.experimental import pallas as pl
from jax.experimental.pallas import tpu as pltpu
from jax.experimental.pallas import tpu_sc as plsc

_NC = 2
_NS = 16
_NW = _NC * _NS
_CHUNK = 128
_NBUF = 8
_K = 4


def _gather_body(table_hbm, idx_hbm, out_hbm, idx_v, rows_v, *sems):
    wid = lax.axis_index("s") * _NC + lax.axis_index("c")
    n_chunks = idx_v.shape[0]
    b_per_w = n_chunks * _CHUNK
    base = wid * b_per_w
    pltpu.sync_copy(idx_hbm.at[wid], idx_v)

    def fire_gather(c, b):
        pltpu.async_copy(table_hbm.at[idx_v.at[c]], rows_v.at[b], sems[b])

    def wait_gather(c, b):
        pltpu.make_async_copy(
            table_hbm.at[idx_v.at[c]], rows_v.at[b], sems[b]).wait()

    def fire_store(c, b):
        pltpu.async_copy(
            rows_v.at[b], out_hbm.at[pl.ds(base + c * _CHUNK, _CHUNK)],
            sems[b])

    def wait_store(b):
        pltpu.make_async_copy(
            rows_v.at[b], out_hbm.at[pl.ds(base, _CHUNK)], sems[b]).wait()

    for c in range(_K):
        fire_gather(c, c % _NBUF)

    for c in range(_NBUF - _K):
        wait_gather(c, c % _NBUF)
        fire_store(c, c % _NBUF)
        fire_gather(c + _K, (c + _K) % _NBUF)

    lo = _NBUF - _K
    hi = n_chunks - _K
    n_groups = (hi - lo) // _NBUF

    def group(i, carry):
        c0 = lo + i * _NBUF
        for j in range(_NBUF):
            c = c0 + j
            b = (lo + j) % _NBUF
            bf = (lo + j + _K) % _NBUF
            wait_gather(c, b)
            fire_store(c, b)
            wait_store(bf)
            fire_gather(c + _K, bf)
        return carry

    lax.fori_loop(0, n_groups, group, 0)

    for c in range(n_chunks - _K, n_chunks):
        wait_gather(c, c % _NBUF)
        fire_store(c, c % _NBUF)

    for b in range(_NBUF):
        wait_store(b)


def kernel(data, indices):
    n, k = indices.shape
    d = data.shape[1]
    b = n * k
    n_chunks = b // (_NW * _CHUNK)
    idx3 = indices.reshape(_NW, n_chunks, _CHUNK).astype(jnp.int32)
    mesh = plsc.VectorSubcoreMesh(core_axis_name="c", subcore_axis_name="s")
    out = pl.kernel(
        _gather_body,
        out_type=jax.ShapeDtypeStruct((b, d), jnp.float32),
        mesh=mesh,
        scratch_types=[
            pltpu.VMEM((n_chunks, _CHUNK), jnp.int32),
            pltpu.VMEM((_NBUF, _CHUNK, d), jnp.float32),
        ] + [pltpu.SemaphoreType.DMA] * _NBUF,
        compiler_params=pltpu.CompilerParams(use_tc_tiling_on_sc=False),
    )(data, idx3)
    return out.reshape(n, k, d)


# speedup vs baseline: 1.9596x; 1.0437x over previous
import jax
import jax.numpy as jnp
from jax import lax
from jax.experimental import pallas as pl
from jax.experimental.pallas import tpu as pltpu
from jax.experimental.pallas import tpu_sc as plsc

_NC = 2
_NS = 16
_NW = _NC * _NS
_CHUNK = 128
_NBUF = 8
_K = 4


def _gather_body(table_hbm, idx_hbm, out_hbm, idx_v, rows_v, *sems):
    wid = lax.axis_index("s") * _NC + lax.axis_index("c")
    n_chunks = idx_v.shape[0]
    b_per_w = n_chunks * _CHUNK
    base = wid * b_per_w
    pltpu.sync_copy(idx_hbm.at[wid], idx_v)

    def fire_gather(c, b):
        pltpu.async_copy(table_hbm.at[idx_v.at[c]], rows_v.at[b], sems[b])

    def wait_gather(c, b):
        pltpu.make_async_copy(
            table_hbm.at[idx_v.at[c]], rows_v.at[b], sems[b]).wait()

    def fire_store(c, b):
        pltpu.async_copy(
            rows_v.at[b], out_hbm.at[pl.ds(base + c * _CHUNK, _CHUNK)],
            sems[b])

    def wait_store(b):
        pltpu.make_async_copy(
            rows_v.at[b], out_hbm.at[pl.ds(base, _CHUNK)], sems[b]).wait()

    for c in range(_K):
        fire_gather(c, c % _NBUF)

    for c in range(_NBUF - _K):
        wait_gather(c, c % _NBUF)
        fire_store(c, c % _NBUF)
        fire_gather(c + _K, (c + _K) % _NBUF)

    lo = _NBUF - _K
    hi = n_chunks - _K
    n_groups = (hi - lo) // _NBUF

    def group(i, carry):
        c0 = lo + i * _NBUF
        for j in range(_NBUF):
            c = c0 + j
            b = (lo + j) % _NBUF
            bf = (lo + j + _K) % _NBUF
            wait_gather(c, b)
            fire_store(c, b)
            wait_store(bf)
            fire_gather(c + _K, bf)
        return carry

    lax.fori_loop(0, n_groups, group, 0)

    for c in range(n_chunks - _K, n_chunks):
        wait_gather(c, c % _NBUF)
        fire_store(c, c % _NBUF)

    for b in range(_NBUF):
        wait_store(b)


def kernel(data, indices):
    n, k = indices.shape
    d = data.shape[1]
    b = n * k
    n_chunks = b // (_NW * _CHUNK)
    idx3 = indices.T.reshape(_NW, n_chunks, _CHUNK).astype(jnp.int32)
    mesh = plsc.VectorSubcoreMesh(core_axis_name="c", subcore_axis_name="s")
    out = pl.kernel(
        _gather_body,
        out_type=jax.ShapeDtypeStruct((b, d), jnp.float32),
        mesh=mesh,
        scratch_types=[
            pltpu.VMEM((n_chunks, _CHUNK), jnp.int32),
            pltpu.VMEM((_NBUF, _CHUNK, d), jnp.float32),
        ] + [pltpu.SemaphoreType.DMA] * _NBUF,
        compiler_params=pltpu.CompilerParams(use_tc_tiling_on_sc=False),
    )(data, idx3)
    return out.reshape(k, n, d).transpose(1, 0, 2)


# speedup vs baseline: 1.9733x; 1.0070x over previous
import jax
import jax.numpy as jnp
from jax import lax
from jax.experimental import pallas as pl
from jax.experimental.pallas import tpu as pltpu
from jax.experimental.pallas import tpu_sc as plsc

_NC = 2
_NS = 16
_NW = _NC * _NS
_CHUNK = 128
_NBUF = 8
_K = 4


def _gather_body(table_hbm, idxt_hbm, out_hbm, idx_v, rows_v, *sems):
    k, n = idxt_hbm.shape
    d = rows_v.shape[2]
    b_per_w = (n * k) // _NW
    n_chunks = b_per_w // _CHUNK
    n_rows = idx_v.shape[0]
    wid = lax.axis_index("s") * _NC + lax.axis_index("c")
    base = wid * b_per_w
    j0 = jnp.minimum(base // n, k - n_rows)
    pltpu.sync_copy(idxt_hbm.at[pl.ds(j0, n_rows)], idx_v)

    def idx_slice(c):
        q = base + c * _CHUNK
        return idx_v.at[q // n - j0, pl.ds(q % n, _CHUNK)]

    def fire_gather(c, b):
        pltpu.async_copy(table_hbm.at[idx_slice(c)], rows_v.at[b], sems[b])

    def wait_gather(b):
        pltpu.make_async_copy(
            table_hbm.at[idx_v.at[0, pl.ds(0, _CHUNK)]], rows_v.at[b],
            sems[b]).wait()

    def fire_store(c, b):
        pltpu.async_copy(
            rows_v.at[b], out_hbm.at[pl.ds(base + c * _CHUNK, _CHUNK)],
            sems[b])

    def wait_store(b):
        pltpu.make_async_copy(
            rows_v.at[b], out_hbm.at[pl.ds(base, _CHUNK)], sems[b]).wait()

    for c in range(_K):
        fire_gather(c, c % _NBUF)

    for c in range(_NBUF - _K):
        wait_gather(c % _NBUF)
        fire_store(c, c % _NBUF)
        fire_gather(c + _K, (c + _K) % _NBUF)

    lo = _NBUF - _K
    hi = n_chunks - _K
    n_groups = (hi - lo) // _NBUF

    def group(i, carry):
        c0 = lo + i * _NBUF
        for j in range(_NBUF):
            c = c0 + j
            b = (lo + j) % _NBUF
            bf = (lo + j + _K) % _NBUF
            wait_gather(b)
            fire_store(c, b)
            wait_store(bf)
            fire_gather(c + _K, bf)
        return carry

    lax.fori_loop(0, n_groups, group, 0)

    for c in range(n_chunks - _K, n_chunks):
        wait_gather(c % _NBUF)
        fire_store(c, c % _NBUF)

    for b in range(_NBUF):
        wait_store(b)


def kernel(data, indices):
    n, k = indices.shape
    d = data.shape[1]
    b = n * k
    idxt = indices.T.astype(jnp.int32)
    n_rows = (b // _NW) // n + 2
    mesh = plsc.VectorSubcoreMesh(core_axis_name="c", subcore_axis_name="s")
    out = pl.kernel(
        _gather_body,
        out_type=jax.ShapeDtypeStruct((b, d), jnp.float32),
        mesh=mesh,
        scratch_types=[
            pltpu.VMEM((n_rows, n), jnp.int32),
            pltpu.VMEM((_NBUF, _CHUNK, d), jnp.float32),
        ] + [pltpu.SemaphoreType.DMA] * _NBUF,
        compiler_params=pltpu.CompilerParams(use_tc_tiling_on_sc=False),
    )(data, idxt)
    return out.reshape(k, n, d).transpose(1, 0, 2)
